# baseline (speedup 1.0000x reference)
import functools

import jax
import jax.numpy as jnp
from jax.experimental import pallas as pl
from jax.experimental.pallas import tpu as pltpu

_BLEND = 0.35
_CONTEXT_BLEND = 0.25

_B = 16
_C = 768
_K = 1024
_KBLK = 128
_NK = _K // _KBLK
_BB = 2


def _pool_body(fm_ref, out_ref):
    out_ref[...] = jnp.sum(fm_ref[...], axis=-1) * (1.0 / fm_ref.shape[-1])


def _l2n(x, eps=1e-6):
    n = jnp.sqrt(jnp.sum(x * x, axis=-1, keepdims=True))
    return x / jnp.maximum(n, eps)


def _fuse_body(pooled_ref, w_ref, gamma_ref, beta_ref, text_ref, proto_ref,
               init_ref, out_ref, ctx_ref):
    k = pl.program_id(0)

    @pl.when(k == 0)
    def _compute_context():
        x = pooled_ref[...]
        y = jnp.dot(x, w_ref[...].T, preferred_element_type=jnp.float32)
        m = jnp.mean(y, axis=-1, keepdims=True)
        v = jnp.mean((y - m) ** 2, axis=-1, keepdims=True)
        y = (y - m) / jnp.sqrt(v + 1e-5) * gamma_ref[...] + beta_ref[...]
        proj = _l2n(y)
        text = text_ref[...]
        bank = jnp.where(init_ref[...] > 0, proto_ref[...], _l2n(text))
        logits = jnp.dot(proj, bank.T, preferred_element_type=jnp.float32)
        mx = jnp.max(logits, axis=-1, keepdims=True)
        e = jnp.exp(logits - mx)
        w = e / jnp.sum(e, axis=-1, keepdims=True)
        ctx_ref[...] = jnp.dot(w, bank, preferred_element_type=jnp.float32)

    ks = k * _KBLK
    text_blk = text_ref[pl.ds(ks, _KBLK), :]
    bank_blk = jnp.where(init_ref[pl.ds(ks, _KBLK), :] > 0,
                         proto_ref[pl.ds(ks, _KBLK), :], _l2n(text_blk))
    enhanced = (1.0 - _BLEND) * text_blk + _BLEND * bank_blk
    ctx = ctx_ref[...]
    pre = ((1.0 - _CONTEXT_BLEND) * enhanced[None, :, :]
           + _CONTEXT_BLEND * ctx[:, None, :])
    out_ref[...] = _l2n(pre)


@jax.jit
def _run(text_features, feature_map, W, gamma, beta, prototype_bank, init_f):
    B, C, H, Wd = feature_map.shape
    fm = feature_map.reshape(B, C, H * Wd)

    fm4 = fm.reshape(B // _BB, _BB, C, H * Wd)
    pooled = pl.pallas_call(
        _pool_body,
        grid=(B // _BB,),
        in_specs=[pl.BlockSpec((1, _BB, C, H * Wd), lambda i: (i, 0, 0, 0))],
        out_specs=pl.BlockSpec((1, _BB, C), lambda i: (i, 0, 0)),
        out_shape=jax.ShapeDtypeStruct((B // _BB, _BB, C), jnp.float32),
    )(fm4).reshape(B, C)

    full = lambda *shape: pl.BlockSpec(shape, lambda k: (0,) * len(shape))
    fused = pl.pallas_call(
        _fuse_body,
        grid=(_NK,),
        in_specs=[
            full(_B, C),
            full(C, C),
            full(1, C),
            full(1, C),
            full(_K, C),
            full(_K, C),
            full(_K, 1),
        ],
        out_specs=pl.BlockSpec((_B, _KBLK, C), lambda k: (0, k, 0)),
        out_shape=jax.ShapeDtypeStruct((_B, _K, C), jnp.float32),
        scratch_shapes=[pltpu.VMEM((_B, C), jnp.float32)],
    )(pooled, W, gamma.reshape(1, C), beta.reshape(1, C),
      text_features, prototype_bank, init_f)
    return fused


def kernel(text_features, feature_map, whwh, W, gamma, beta, prototype_bank,
           prototype_initialized):
    del whwh
    init_f = prototype_initialized.astype(jnp.float32).reshape(-1, 1)
    return _run(text_features, feature_map, W, gamma, beta, prototype_bank,
                init_f)
